# baseline (speedup 1.0000x reference)
import functools

import jax
import jax.numpy as jnp
from jax import lax
from jax.experimental import pallas as pl
from jax.experimental.pallas import tpu as pltpu
from jax.experimental.pallas import tpu_sc as plsc

_NC = 2
_NS = 16
_L = 16
_B = 8
_C = 128
_FRAC = 1.0


def _nbf(n):
    return int(n // _B * _FRAC) // 8 * 8


def _layout(s, sp):
    wt = _NC * _NS
    assert s % 8 == 0
    q = (s // wt) // 8 * 8
    r = (s - q * wt) // 8
    swmax = q + (8 if r else 0)
    garb = swmax
    accr = ((swmax + 2 + _L - 1) // _L) * _L
    ow = ((swmax + 1 + 7) // 8) * 8
    assert ow <= sp
    steps = []
    st = 1
    while st < ow:
        st *= 2
    while st >= 1:
        steps.append(st)
        st //= 2
    return q, r, swmax, garb, accr, ow, steps


def _searcher(offs_l, ow, steps):
    def search(g):
        pos = jnp.zeros((_L,), jnp.int32)
        for stp in steps:
            cand = pos + stp
            cc = jnp.minimum(cand, ow - 1)
            v = plsc.load_gather(offs_l, [cc])
            pos = jnp.where((cand <= ow - 1) & (v <= g), cand, pos)
        return pos
    return search


@functools.lru_cache(maxsize=None)
def _build_bs(nb, nbf, d):
    tb = 512
    while nbf % tb or tb % 8:
        tb -= 1

    def body(x_ref, o_ref):
        ones = jnp.ones((_B,), jnp.float32)
        o_ref[...] = jax.lax.dot_general(
            x_ref[...], ones, (((1,), (0,)), ((), ())),
            preferred_element_type=jnp.float32)

    return pl.pallas_call(
        body,
        grid=(nbf // tb,),
        in_specs=[pl.BlockSpec((tb, _B, d), lambda i: (i, 0, 0))],
        out_specs=pl.BlockSpec((tb, d), lambda i: (i, 0)),
        out_shape=jax.ShapeDtypeStruct((nbf, d), jnp.float32),
    )


@functools.lru_cache(maxsize=None)
def _build_main(n, d, s, sp):
    q, r, swmax, garb, accr, ow, steps = _layout(s, sp)
    nbf = _nbf(n)
    spl = nbf * _B
    ng = (swmax + 1 + _L - 1) // _L
    mesh = plsc.VectorSubcoreMesh(core_axis_name="c", subcore_axis_name="s")

    @functools.partial(
        pl.kernel,
        out_type=jax.ShapeDtypeStruct((s, d), jnp.float32),
        mesh=mesh,
        scratch_types=[
            pltpu.VMEM((ow,), jnp.int32),
            [pltpu.VMEM((_C, d), jnp.float32) for _ in range(2)],
            [pltpu.VMEM((_C,), jnp.int32) for _ in range(2)],
            [pltpu.VMEM((_C,), jnp.int32) for _ in range(2)],
            [pltpu.VMEM((_L,), jnp.int32) for _ in range(2)],
            pltpu.VMEM((_L, d), jnp.float32),
            pltpu.VMEM_SHARED((_NS * accr, d), jnp.float32),
            [pltpu.SemaphoreType.DMA for _ in range(2)],
        ],
        compiler_params=pltpu.CompilerParams(needs_layout_passes=False),
    )
    def main_kernel(data_hbm, offs_hbm, outm_hbm, offs_l, bufs, idxbs,
                    ridxs, kbufs, zbuf, acc, gsems):
        cid = lax.axis_index("c")
        sid = lax.axis_index("s")
        w = sid * _NC + cid
        s0 = w * q + jnp.minimum(w, r) * 8
        nseg = jnp.where(w < r, q + 8, q)
        abase = sid * accr

        for rr in range(_L):
            for cc in range(d // _L):
                zbuf[rr, pl.ds(cc * _L, _L)] = jnp.zeros((_L,), jnp.float32)
        for t in range(accr // _L):
            pltpu.sync_copy(zbuf, acc.at[pl.ds(abase + t * _L, _L)])

        base_a = jnp.minimum(s0, sp - ow)
        pltpu.sync_copy(offs_hbm.at[pl.ds(base_a, ow)], offs_l)
        search = _searcher(offs_l, ow, steps)
        lanes = lax.iota(jnp.int32, _L)

        def _scalar_at(i):
            return plsc.load_gather(
                offs_l, [jnp.full((_L,), i, jnp.int32)])[0]

        rs = _scalar_at(s0 - base_a)
        re = _scalar_at(s0 + nseg - base_a)

        def ebuild(gi, bn):
            i = gi * _L + lanes
            inb = i <= nseg
            iw = jnp.minimum(s0 - base_a + i, ow - 1)
            bval = plsc.load_gather(offs_l, [iw])
            bvalp = plsc.load_gather(offs_l, [jnp.maximum(iw - 1, 0)])
            blk = bval // _B
            strad = ((bval % _B) != 0) & (blk < nbf)
            dup = (blk == bvalp // _B) & ((bvalp % _B) != 0) & (i > 0)
            keep = inb & strad & jnp.logical_not(dup)
            kbufs[bn][...] = keep.astype(jnp.int32)
            blkc = jnp.minimum(blk, nbf - 1)
            for j in range(_B):
                plsc.store_scatter(ridxs[bn], [lanes * _B + j],
                                   blkc * _B + j)

            @pl.when(plsc.all_reduce_population_count(keep)[0] > 0)
            def _():
                pltpu.make_async_copy(
                    data_hbm.at[ridxs[bn]], bufs[bn], gsems[bn]).start()

        def eprocess(gi, bn):
            kv = kbufs[bn][...] != 0

            @pl.when(plsc.all_reduce_population_count(kv)[0] > 0)
            def _():
                pltpu.make_async_copy(
                    data_hbm.at[ridxs[bn]], bufs[bn], gsems[bn]).wait()
                for k in range(_C // _L):
                    g = ridxs[bn][pl.ds(k * _L, _L)]
                    seg = base_a + search(g)
                    kr = plsc.load_gather(
                        kbufs[bn], [(lanes // _B) + (_L // _B) * k])
                    valid = ((kr != 0) & (g >= rs)
                             & (seg >= s0) & (seg < s0 + nseg))
                    idxbs[bn][pl.ds(k * _L, _L)] = (
                        jnp.where(valid, seg - s0, garb) + abase)
                pltpu.sync_copy(bufs[bn], acc.at[idxbs[bn]], add=True)

        ebuild(0, 0)

        def epair(c2, carry):
            for b in range(2):
                gg = 2 * c2 + b

                @pl.when(gg < ng)
                def _(gg=gg, b=b):
                    @pl.when(gg + 1 < ng)
                    def _():
                        ebuild(gg + 1, 1 - b)

                    eprocess(gg, b)
            return carry

        lax.fori_loop(0, (ng + 1) // 2, epair, 0)

        rs_m = jnp.maximum(rs, spl)
        rs8 = (rs_m // 8) * 8
        nch = jnp.maximum(re - rs8 + (_C - 1), 0) // _C

        def _gather(c, buf, sem):
            base = rs8 + c * _C
            cb = jnp.minimum(base, n - _C)
            return pltpu.make_async_copy(data_hbm.at[pl.ds(cb, _C)],
                                         buf, sem)

        def _process(c, bn):
            base = rs8 + c * _C
            cb = jnp.minimum(base, n - _C)
            for gi in range(_C // _L):
                g = cb + gi * _L + lanes
                valid = (g >= jnp.maximum(base, rs_m)) & (g < re)
                seg = base_a + search(g)
                idxbs[bn][pl.ds(gi * _L, _L)] = (
                    jnp.where(valid, seg - s0, garb) + abase)
            pltpu.sync_copy(bufs[bn], acc.at[idxbs[bn]], add=True)

        @pl.when(nch > 0)
        def _():
            _gather(0, bufs[0], gsems[0]).start()

        def chunk_pair(c2, carry):
            for b in range(2):
                cc = 2 * c2 + b

                @pl.when(cc < nch)
                def _(cc=cc, b=b):
                    _gather(cc, bufs[b], gsems[b]).wait()

                    @pl.when(cc + 1 < nch)
                    def _():
                        _gather(cc + 1, bufs[1 - b], gsems[1 - b]).start()

                    _process(cc, b)
            return carry

        lax.fori_loop(0, (nch + 1) // 2, chunk_pair, 0)

        if r:
            @pl.when(w < r)
            def _():
                pltpu.sync_copy(acc.at[pl.ds(abase, q + 8)],
                                outm_hbm.at[pl.ds(s0, q + 8)])

        if q:
            @pl.when(w >= r)
            def _():
                pltpu.sync_copy(acc.at[pl.ds(abase, q)],
                                outm_hbm.at[pl.ds(s0, q)])

    return main_kernel


@functools.lru_cache(maxsize=None)
def _build_blocks(n, d, s, sp):
    q, r, swmax, garb, accr, ow, steps = _layout(s, sp)
    nbf = _nbf(n)
    mesh = plsc.VectorSubcoreMesh(core_axis_name="c", subcore_axis_name="s")

    @functools.partial(
        pl.kernel,
        out_type=jax.ShapeDtypeStruct((s, d), jnp.float32),
        mesh=mesh,
        scratch_types=[
            pltpu.VMEM((ow,), jnp.int32),
            [pltpu.VMEM((_C, d), jnp.float32) for _ in range(2)],
            [pltpu.VMEM((_C,), jnp.int32) for _ in range(2)],
            pltpu.VMEM((_L, d), jnp.float32),
            pltpu.VMEM_SHARED((_NS * accr, d), jnp.float32),
            [pltpu.SemaphoreType.DMA for _ in range(2)],
        ],
        compiler_params=pltpu.CompilerParams(needs_layout_passes=False),
    )
    def block_kernel(bs_hbm, offs_hbm, outm_hbm, out_hbm, offs_l, bufs,
                     idxbs, zbuf, acc, gsems):
        cid = lax.axis_index("c")
        sid = lax.axis_index("s")
        w = sid * _NC + cid
        s0 = w * q + jnp.minimum(w, r) * 8
        nseg = jnp.where(w < r, q + 8, q)
        abase = sid * accr

        for rr in range(_L):
            for cc in range(d // _L):
                zbuf[rr, pl.ds(cc * _L, _L)] = jnp.zeros((_L,), jnp.float32)
        if r:
            @pl.when(w < r)
            def _():
                for t in range((q + 8) // _L, accr // _L):
                    pltpu.sync_copy(zbuf, acc.at[pl.ds(abase + t * _L, _L)])
                pltpu.sync_copy(outm_hbm.at[pl.ds(s0, q + 8)],
                                acc.at[pl.ds(abase, q + 8)])
        if q:
            @pl.when(w >= r)
            def _():
                for t in range(q // _L, accr // _L):
                    pltpu.sync_copy(zbuf, acc.at[pl.ds(abase + t * _L, _L)])
                pltpu.sync_copy(outm_hbm.at[pl.ds(s0, q)],
                                acc.at[pl.ds(abase, q)])

        base_a = jnp.minimum(s0, sp - ow)
        pltpu.sync_copy(offs_hbm.at[pl.ds(base_a, ow)], offs_l)
        search = _searcher(offs_l, ow, steps)
        lanes = lax.iota(jnp.int32, _L)

        def _scalar_at(i):
            return plsc.load_gather(
                offs_l, [jnp.full((_L,), i, jnp.int32)])[0]

        rs = _scalar_at(s0 - base_a)
        re = _scalar_at(s0 + nseg - base_a)
        blo = (rs + _B - 1) // _B
        bhi = jnp.minimum(re // _B, nbf)
        b8 = (blo // 8) * 8
        nch = jnp.maximum(bhi - b8 + (_C - 1), 0) // _C

        def _gather(c, buf, sem):
            base = b8 + c * _C
            cb = jnp.minimum(base, nbf - _C)
            return pltpu.make_async_copy(bs_hbm.at[pl.ds(cb, _C)], buf, sem)

        def _process(c, bn):
            base = b8 + c * _C
            cb = jnp.minimum(base, nbf - _C)
            for gj in range(_C // _L):
                b = cb + gj * _L + lanes
                valid = (b >= jnp.maximum(base, blo)) & (b < bhi)
                pos1 = search(b * _B)
                nv = plsc.load_gather(
                    offs_l, [jnp.minimum(pos1 + 1, ow - 1)])
                interior = nv > b * _B + (_B - 1)
                idxbs[bn][pl.ds(gj * _L, _L)] = (
                    jnp.where(valid & interior,
                              base_a + pos1 - s0, garb) + abase)
            pltpu.sync_copy(bufs[bn], acc.at[idxbs[bn]], add=True)

        @pl.when(nch > 0)
        def _():
            _gather(0, bufs[0], gsems[0]).start()

        def chunk_pair(c2, carry):
            for bnum in range(2):
                cc = 2 * c2 + bnum

                @pl.when(cc < nch)
                def _(cc=cc, bnum=bnum):
                    _gather(cc, bufs[bnum], gsems[bnum]).wait()

                    @pl.when(cc + 1 < nch)
                    def _():
                        _gather(cc + 1, bufs[1 - bnum],
                                gsems[1 - bnum]).start()

                    _process(cc, bnum)
            return carry

        lax.fori_loop(0, (nch + 1) // 2, chunk_pair, 0)

        if r:
            @pl.when(w < r)
            def _():
                pltpu.sync_copy(acc.at[pl.ds(abase, q + 8)],
                                out_hbm.at[pl.ds(s0, q + 8)])

        if q:
            @pl.when(w >= r)
            def _():
                pltpu.sync_copy(acc.at[pl.ds(abase, q)],
                                out_hbm.at[pl.ds(s0, q)])

    return block_kernel


def kernel(data, offsets):
    n, d = data.shape
    s = offsets.shape[0] - 1
    offs = offsets.astype(jnp.int32)
    pad = (-offsets.shape[0]) % 8
    if pad:
        offs = jnp.concatenate([offs, jnp.full((pad,), n, jnp.int32)])
    sp = int(offs.shape[0])
    outm = _build_main(n, d, s, sp)(data, offs)
    bs = _build_bs(n // _B, _nbf(n), d)(data.reshape(n // _B, _B, d))
    return _build_blocks(n, d, s, sp)(bs, offs, outm)


# speedup vs baseline: 9.2461x; 9.2461x over previous
import functools

import jax
import jax.numpy as jnp
from jax import lax
from jax.experimental import pallas as pl
from jax.experimental.pallas import tpu as pltpu
from jax.experimental.pallas import tpu_sc as plsc

_NC = 2
_NS = 16
_L = 16
_B = 8
_C = 128
_FRAC = 1.0


def _nbf(n):
    return int(n // _B * _FRAC) // 8 * 8


def _layout(s, sp):
    wt = _NC * _NS
    assert s % 8 == 0
    q = (s // wt) // 8 * 8
    r = (s - q * wt) // 8
    swmax = q + (8 if r else 0)
    garb = swmax
    accr = ((swmax + 2 + _L - 1) // _L) * _L
    ow = ((swmax + 1 + 7) // 8) * 8
    assert ow <= sp
    steps = []
    st = 1
    while st < ow:
        st *= 2
    while st >= 1:
        steps.append(st)
        st //= 2
    return q, r, swmax, garb, accr, ow, steps


def _searcher(offs_l, ow, steps):
    def search(g):
        pos = jnp.zeros((_L,), jnp.int32)
        for stp in steps:
            cand = pos + stp
            cc = jnp.minimum(cand, ow - 1)
            v = plsc.load_gather(offs_l, [cc])
            pos = jnp.where((cand <= ow - 1) & (v <= g), cand, pos)
        return pos
    return search


@functools.lru_cache(maxsize=None)
def _build_bs(nb, nbf, d):
    tb = 512
    while nbf % tb or tb % 8:
        tb -= 1

    def body(x_ref, o_ref):
        o_ref[...] = jnp.sum(x_ref[...], axis=1)

    return pl.pallas_call(
        body,
        grid=(nbf // tb,),
        in_specs=[pl.BlockSpec((tb, _B, d), lambda i: (i, 0, 0))],
        out_specs=pl.BlockSpec((tb, d), lambda i: (i, 0)),
        out_shape=jax.ShapeDtypeStruct((nbf, d), jnp.float32),
    )


@functools.lru_cache(maxsize=None)
def _build_main(n, d, s, sp):
    q, r, swmax, garb, accr, ow, steps = _layout(s, sp)
    nbf = _nbf(n)
    spl = nbf * _B
    ng = (swmax + 1 + _L - 1) // _L
    mesh = plsc.VectorSubcoreMesh(core_axis_name="c", subcore_axis_name="s")

    @functools.partial(
        pl.kernel,
        out_type=jax.ShapeDtypeStruct((s, d), jnp.float32),
        mesh=mesh,
        scratch_types=[
            pltpu.VMEM((ow,), jnp.int32),
            [pltpu.VMEM((_C, d), jnp.float32) for _ in range(2)],
            [pltpu.VMEM((_C,), jnp.int32) for _ in range(2)],
            [pltpu.VMEM((_C,), jnp.int32) for _ in range(2)],
            [pltpu.VMEM((_L,), jnp.int32) for _ in range(2)],
            pltpu.VMEM((_L, d), jnp.float32),
            pltpu.VMEM_SHARED((_NS * accr, d), jnp.float32),
            [pltpu.SemaphoreType.DMA for _ in range(2)],
        ],
        compiler_params=pltpu.CompilerParams(needs_layout_passes=False),
    )
    def main_kernel(data_hbm, offs_hbm, outm_hbm, offs_l, bufs, idxbs,
                    ridxs, kbufs, zbuf, acc, gsems):
        cid = lax.axis_index("c")
        sid = lax.axis_index("s")
        w = sid * _NC + cid
        s0 = w * q + jnp.minimum(w, r) * 8
        nseg = jnp.where(w < r, q + 8, q)
        abase = sid * accr

        for rr in range(_L):
            for cc in range(d // _L):
                zbuf[rr, pl.ds(cc * _L, _L)] = jnp.zeros((_L,), jnp.float32)
        for t in range(accr // _L):
            pltpu.sync_copy(zbuf, acc.at[pl.ds(abase + t * _L, _L)])

        base_a = jnp.minimum(s0, sp - ow)
        pltpu.sync_copy(offs_hbm.at[pl.ds(base_a, ow)], offs_l)
        search = _searcher(offs_l, ow, steps)
        lanes = lax.iota(jnp.int32, _L)

        def _scalar_at(i):
            return plsc.load_gather(
                offs_l, [jnp.full((_L,), i, jnp.int32)])[0]

        rs = _scalar_at(s0 - base_a)
        re = _scalar_at(s0 + nseg - base_a)

        def ebuild(gi, bn):
            i = gi * _L + lanes
            inb = i <= nseg
            iw = jnp.minimum(s0 - base_a + i, ow - 1)
            bval = plsc.load_gather(offs_l, [iw])
            bvalp = plsc.load_gather(offs_l, [jnp.maximum(iw - 1, 0)])
            blk = bval // _B
            strad = ((bval % _B) != 0) & (blk < nbf)
            dup = (blk == bvalp // _B) & ((bvalp % _B) != 0) & (i > 0)
            keep = inb & strad & jnp.logical_not(dup)
            kbufs[bn][...] = keep.astype(jnp.int32)
            blkc = jnp.minimum(blk, nbf - 1)
            for j in range(_B):
                plsc.store_scatter(ridxs[bn], [lanes * _B + j],
                                   blkc * _B + j)

            @pl.when(plsc.all_reduce_population_count(keep)[0] > 0)
            def _():
                pltpu.make_async_copy(
                    data_hbm.at[ridxs[bn]], bufs[bn], gsems[bn]).start()

        def eprocess(gi, bn):
            kv = kbufs[bn][...] != 0

            @pl.when(plsc.all_reduce_population_count(kv)[0] > 0)
            def _():
                pltpu.make_async_copy(
                    data_hbm.at[ridxs[bn]], bufs[bn], gsems[bn]).wait()
                for k in range(_C // _L):
                    g = ridxs[bn][pl.ds(k * _L, _L)]
                    seg = base_a + search(g)
                    kr = plsc.load_gather(
                        kbufs[bn], [(lanes // _B) + (_L // _B) * k])
                    valid = ((kr != 0) & (g >= rs)
                             & (seg >= s0) & (seg < s0 + nseg))
                    idxbs[bn][pl.ds(k * _L, _L)] = (
                        jnp.where(valid, seg - s0, garb) + abase)
                pltpu.sync_copy(bufs[bn], acc.at[idxbs[bn]], add=True)

        ebuild(0, 0)

        def epair(c2, carry):
            for b in range(2):
                gg = 2 * c2 + b

                @pl.when(gg < ng)
                def _(gg=gg, b=b):
                    @pl.when(gg + 1 < ng)
                    def _():
                        ebuild(gg + 1, 1 - b)

                    eprocess(gg, b)
            return carry

        lax.fori_loop(0, (ng + 1) // 2, epair, 0)

        rs_m = jnp.maximum(rs, spl)
        rs8 = (rs_m // 8) * 8
        nch = jnp.maximum(re - rs8 + (_C - 1), 0) // _C

        def _gather(c, buf, sem):
            base = rs8 + c * _C
            cb = jnp.minimum(base, n - _C)
            return pltpu.make_async_copy(data_hbm.at[pl.ds(cb, _C)],
                                         buf, sem)

        def _process(c, bn):
            base = rs8 + c * _C
            cb = jnp.minimum(base, n - _C)
            for gi in range(_C // _L):
                g = cb + gi * _L + lanes
                valid = (g >= jnp.maximum(base, rs_m)) & (g < re)
                seg = base_a + search(g)
                idxbs[bn][pl.ds(gi * _L, _L)] = (
                    jnp.where(valid, seg - s0, garb) + abase)
            pltpu.sync_copy(bufs[bn], acc.at[idxbs[bn]], add=True)

        @pl.when(nch > 0)
        def _():
            _gather(0, bufs[0], gsems[0]).start()

        def chunk_pair(c2, carry):
            for b in range(2):
                cc = 2 * c2 + b

                @pl.when(cc < nch)
                def _(cc=cc, b=b):
                    _gather(cc, bufs[b], gsems[b]).wait()

                    @pl.when(cc + 1 < nch)
                    def _():
                        _gather(cc + 1, bufs[1 - b], gsems[1 - b]).start()

                    _process(cc, b)
            return carry

        lax.fori_loop(0, (nch + 1) // 2, chunk_pair, 0)

        if r:
            @pl.when(w < r)
            def _():
                pltpu.sync_copy(acc.at[pl.ds(abase, q + 8)],
                                outm_hbm.at[pl.ds(s0, q + 8)])

        if q:
            @pl.when(w >= r)
            def _():
                pltpu.sync_copy(acc.at[pl.ds(abase, q)],
                                outm_hbm.at[pl.ds(s0, q)])

    return main_kernel


@functools.lru_cache(maxsize=None)
def _build_blocks(n, d, s, sp):
    q, r, swmax, garb, accr, ow, steps = _layout(s, sp)
    nbf = _nbf(n)
    mesh = plsc.VectorSubcoreMesh(core_axis_name="c", subcore_axis_name="s")

    @functools.partial(
        pl.kernel,
        out_type=jax.ShapeDtypeStruct((s, d), jnp.float32),
        mesh=mesh,
        scratch_types=[
            pltpu.VMEM((ow,), jnp.int32),
            [pltpu.VMEM((_C, d), jnp.float32) for _ in range(2)],
            [pltpu.VMEM((_C,), jnp.int32) for _ in range(2)],
            pltpu.VMEM((_L, d), jnp.float32),
            pltpu.VMEM_SHARED((_NS * accr, d), jnp.float32),
            [pltpu.SemaphoreType.DMA for _ in range(2)],
        ],
        compiler_params=pltpu.CompilerParams(needs_layout_passes=False),
    )
    def block_kernel(bs_hbm, offs_hbm, outm_hbm, out_hbm, offs_l, bufs,
                     idxbs, zbuf, acc, gsems):
        cid = lax.axis_index("c")
        sid = lax.axis_index("s")
        w = sid * _NC + cid
        s0 = w * q + jnp.minimum(w, r) * 8
        nseg = jnp.where(w < r, q + 8, q)
        abase = sid * accr

        for rr in range(_L):
            for cc in range(d // _L):
                zbuf[rr, pl.ds(cc * _L, _L)] = jnp.zeros((_L,), jnp.float32)
        if r:
            @pl.when(w < r)
            def _():
                for t in range((q + 8) // _L, accr // _L):
                    pltpu.sync_copy(zbuf, acc.at[pl.ds(abase + t * _L, _L)])
                pltpu.sync_copy(outm_hbm.at[pl.ds(s0, q + 8)],
                                acc.at[pl.ds(abase, q + 8)])
        if q:
            @pl.when(w >= r)
            def _():
                for t in range(q // _L, accr // _L):
                    pltpu.sync_copy(zbuf, acc.at[pl.ds(abase + t * _L, _L)])
                pltpu.sync_copy(outm_hbm.at[pl.ds(s0, q)],
                                acc.at[pl.ds(abase, q)])

        base_a = jnp.minimum(s0, sp - ow)
        pltpu.sync_copy(offs_hbm.at[pl.ds(base_a, ow)], offs_l)
        search = _searcher(offs_l, ow, steps)
        lanes = lax.iota(jnp.int32, _L)

        def _scalar_at(i):
            return plsc.load_gather(
                offs_l, [jnp.full((_L,), i, jnp.int32)])[0]

        rs = _scalar_at(s0 - base_a)
        re = _scalar_at(s0 + nseg - base_a)
        blo = (rs + _B - 1) // _B
        bhi = jnp.minimum(re // _B, nbf)
        b8 = (blo // 8) * 8
        nch = jnp.maximum(bhi - b8 + (_C - 1), 0) // _C

        def _gather(c, buf, sem):
            base = b8 + c * _C
            cb = jnp.minimum(base, nbf - _C)
            return pltpu.make_async_copy(bs_hbm.at[pl.ds(cb, _C)], buf, sem)

        def _process(c, bn):
            base = b8 + c * _C
            cb = jnp.minimum(base, nbf - _C)
            for gj in range(_C // _L):
                b = cb + gj * _L + lanes
                valid = (b >= jnp.maximum(base, blo)) & (b < bhi)
                pos1 = search(b * _B)
                nv = plsc.load_gather(
                    offs_l, [jnp.minimum(pos1 + 1, ow - 1)])
                interior = nv > b * _B + (_B - 1)
                idxbs[bn][pl.ds(gj * _L, _L)] = (
                    jnp.where(valid & interior,
                              base_a + pos1 - s0, garb) + abase)
            pltpu.sync_copy(bufs[bn], acc.at[idxbs[bn]], add=True)

        @pl.when(nch > 0)
        def _():
            _gather(0, bufs[0], gsems[0]).start()

        def chunk_pair(c2, carry):
            for bnum in range(2):
                cc = 2 * c2 + bnum

                @pl.when(cc < nch)
                def _(cc=cc, bnum=bnum):
                    _gather(cc, bufs[bnum], gsems[bnum]).wait()

                    @pl.when(cc + 1 < nch)
                    def _():
                        _gather(cc + 1, bufs[1 - bnum],
                                gsems[1 - bnum]).start()

                    _process(cc, bnum)
            return carry

        lax.fori_loop(0, (nch + 1) // 2, chunk_pair, 0)

        if r:
            @pl.when(w < r)
            def _():
                pltpu.sync_copy(acc.at[pl.ds(abase, q + 8)],
                                out_hbm.at[pl.ds(s0, q + 8)])

        if q:
            @pl.when(w >= r)
            def _():
                pltpu.sync_copy(acc.at[pl.ds(abase, q)],
                                out_hbm.at[pl.ds(s0, q)])

    return block_kernel


def kernel(data, offsets):
    n, d = data.shape
    s = offsets.shape[0] - 1
    offs = offsets.astype(jnp.int32)
    pad = (-offsets.shape[0]) % 8
    if pad:
        offs = jnp.concatenate([offs, jnp.full((pad,), n, jnp.int32)])
    sp = int(offs.shape[0])
    outm = _build_main(n, d, s, sp)(data, offs)
    bs = _build_bs(n // _B, _nbf(n), d)(data.reshape(n // _B, _B, d))
    return _build_blocks(n, d, s, sp)(bs, offs, outm)


# speedup vs baseline: 9.3382x; 1.0100x over previous
import functools

import jax
import jax.numpy as jnp
from jax import lax
from jax.experimental import pallas as pl
from jax.experimental.pallas import tpu as pltpu
from jax.experimental.pallas import tpu_sc as plsc

_NC = 2
_NS = 16
_L = 16
_C = 256
_CS = 128
_NB = 2


@functools.lru_cache(maxsize=None)
def _build(n, d, s, sp):
    w_total = _NC * _NS
    assert s % 8 == 0 and n % 8 == 0
    q = (s // w_total) // 8 * 8
    r = (s - q * w_total) // 8
    swmax = q + (8 if r else 0)
    garb = swmax
    accr = ((swmax + 2 + _L - 1) // _L) * _L
    ow = ((swmax + 1 + 7) // 8) * 8
    assert ow <= sp
    steps = []
    st = 1
    while st < ow:
        st *= 2
    while st >= 1:
        steps.append(st)
        st //= 2

    mesh = plsc.VectorSubcoreMesh(core_axis_name="c", subcore_axis_name="s")

    @functools.partial(
        pl.kernel,
        out_type=jax.ShapeDtypeStruct((s, d), jnp.float32),
        mesh=mesh,
        scratch_types=[
            pltpu.VMEM((ow,), jnp.int32),
            [pltpu.VMEM((_C, d), jnp.float32) for _ in range(_NB)],
            [pltpu.VMEM((_CS,), jnp.int32) for _ in range(_C // _CS)],
            pltpu.VMEM((_L, d), jnp.float32),
            pltpu.VMEM_SHARED((_NS * accr, d), jnp.float32),
            [pltpu.SemaphoreType.DMA for _ in range(_NB)],
        ],
        compiler_params=pltpu.CompilerParams(needs_layout_passes=False),
    )
    def seg_kernel(data_hbm, offs_hbm, out_hbm, offs_l, bufs, idxbs,
                   zbuf, acc, gsems):
        cid = lax.axis_index("c")
        sid = lax.axis_index("s")
        w = cid * _NS + sid
        s0 = w * q + jnp.minimum(w, r) * 8
        nseg = jnp.where(w < r, q + 8, q)
        abase = sid * accr

        for rr in range(_L):
            for cc in range(d // _L):
                zbuf[rr, pl.ds(cc * _L, _L)] = jnp.zeros((_L,), jnp.float32)
        for t in range(accr // _L):
            pltpu.sync_copy(zbuf, acc.at[pl.ds(abase + t * _L, _L)])

        base_a = jnp.minimum(s0, sp - ow)
        pltpu.sync_copy(offs_hbm.at[pl.ds(base_a, ow)], offs_l)
        def _scalar_at(i):
            return plsc.load_gather(
                offs_l, [jnp.full((_L,), i, jnp.int32)])[0]

        rs = _scalar_at(s0 - base_a)
        re = _scalar_at(s0 + nseg - base_a)

        rs8 = (rs // 8) * 8
        nch = (re - rs8 + (_C - 1)) // _C

        def _gather(c, buf, sem):
            base = rs8 + c * _C
            cb = jnp.minimum(base, n - _C)
            return pltpu.make_async_copy(data_hbm.at[pl.ds(cb, _C)], buf, sem)

        def _process(c, buf):
            base = rs8 + c * _C
            cb = jnp.minimum(base, n - _C)
            for h in range(_C // _CS):
                idxb = idxbs[h]
                for gi in range(_CS // _L):
                    g = cb + h * _CS + gi * _L + lax.iota(jnp.int32, _L)
                    valid = (g >= jnp.maximum(base, rs)) & (g < re)
                    pos = jnp.zeros((_L,), jnp.int32)
                    for stp in steps:
                        cand = pos + stp
                        cc2 = jnp.minimum(cand, ow - 1)
                        v = plsc.load_gather(offs_l, [cc2])
                        pos = jnp.where((cand <= ow - 1) & (v <= g),
                                        cand, pos)
                    aidx = base_a + pos - s0
                    idxb[pl.ds(gi * _L, _L)] = (
                        jnp.where(valid, aidx, garb) + abase)
                pltpu.sync_copy(buf.at[pl.ds(h * _CS, _CS)],
                                acc.at[idxb], add=True)

        for b in range(_NB - 1):
            @pl.when(b < nch)
            def _(b=b):
                _gather(b, bufs[b], gsems[b]).start()

        def chunk_group(cg, carry):
            for b in range(_NB):
                cc = _NB * cg + b

                @pl.when(cc < nch)
                def _(cc=cc, b=b):
                    _gather(cc, bufs[b], gsems[b]).wait()
                    nb = (b + _NB - 1) % _NB

                    @pl.when(cc + _NB - 1 < nch)
                    def _():
                        _gather(cc + _NB - 1, bufs[nb], gsems[nb]).start()

                    _process(cc, bufs[b])
            return carry

        lax.fori_loop(0, (nch + _NB - 1) // _NB, chunk_group, 0)

        if r:
            @pl.when(w < r)
            def _():
                pltpu.sync_copy(acc.at[pl.ds(abase, q + 8)],
                                out_hbm.at[pl.ds(s0, q + 8)])

        if q:
            @pl.when(w >= r)
            def _():
                pltpu.sync_copy(acc.at[pl.ds(abase, q)],
                                out_hbm.at[pl.ds(s0, q)])

    return seg_kernel


def kernel(data, offsets):
    n, d = data.shape
    s = offsets.shape[0] - 1
    offs = offsets.astype(jnp.int32)
    pad = (-offsets.shape[0]) % 8
    if pad:
        offs = jnp.concatenate([offs, jnp.full((pad,), n, jnp.int32)])
    return _build(n, d, s, int(offs.shape[0]))(data, offs)
